# baseline (device time: 69182 ns/iter reference)
import jax
import jax.numpy as jnp
from jax import lax
from jax.experimental import pallas as pl
from jax.experimental.pallas import tpu as pltpu

N_DEV = 8
N_STEPS = 3

_sem_signal = getattr(pltpu, "semaphore_signal", None) or getattr(pl, "semaphore_signal")
_sem_wait = getattr(pltpu, "semaphore_wait", None) or getattr(pl, "semaphore_wait")
_DeviceIdType = getattr(pl, "DeviceIdType", None) or getattr(pltpu, "DeviceIdType")
_CompilerParams = getattr(pltpu, "CompilerParams", None) or getattr(
    pltpu, "TPUCompilerParams"
)


def _layer(x, win, wout, cid):
    b, d = x.shape

    def body(x_ref, win_ref, wout_ref, out_ref, sendbuf, recvbuf, send_sems, recv_sems):
        my = lax.axis_index("i")

        barrier = pltpu.get_barrier_semaphore()
        for k in range(N_STEPS):
            partner = my ^ (1 << k)
            _sem_signal(
                barrier,
                inc=1,
                device_id=(partner,),
                device_id_type=_DeviceIdType.MESH,
            )
        _sem_wait(barrier, N_STEPS)

        h = jnp.maximum(
            jnp.dot(x_ref[:, :], win_ref[:, :], preferred_element_type=jnp.float32),
            0.0,
        )
        acc = jnp.dot(h, wout_ref[:, :], preferred_element_type=jnp.float32)

        for k in range(N_STEPS):
            partner = my ^ (1 << k)
            sendbuf[:, :] = acc
            rdma = pltpu.make_async_remote_copy(
                src_ref=sendbuf,
                dst_ref=recvbuf.at[k],
                send_sem=send_sems.at[k],
                recv_sem=recv_sems.at[k],
                device_id=(partner,),
                device_id_type=_DeviceIdType.MESH,
            )
            rdma.start()
            rdma.wait()
            acc = acc + recvbuf[k, :, :]

        out_ref[:, :] = acc

    return pl.pallas_call(
        body,
        out_shape=jax.ShapeDtypeStruct((b, d), jnp.float32),
        in_specs=[
            pl.BlockSpec(memory_space=pltpu.VMEM),
            pl.BlockSpec(memory_space=pltpu.VMEM),
            pl.BlockSpec(memory_space=pltpu.VMEM),
        ],
        out_specs=pl.BlockSpec(memory_space=pltpu.VMEM),
        scratch_shapes=[
            pltpu.VMEM((b, d), jnp.float32),
            pltpu.VMEM((N_STEPS, b, d), jnp.float32),
            pltpu.SemaphoreType.DMA((N_STEPS,)),
            pltpu.SemaphoreType.DMA((N_STEPS,)),
        ],
        compiler_params=_CompilerParams(collective_id=cid),
    )(x, win, wout)


def kernel(x, Win0, Wout0, Win1, Wout1, Win2, Wout2):
    x = _layer(x, Win0, Wout0, 0)
    x = _layer(x, Win1, Wout1, 1)
    x = _layer(x, Win2, Wout2, 2)
    return x


# device time: 21085 ns/iter; 3.2811x vs baseline; 3.2811x over previous
import jax
import jax.numpy as jnp
from jax import lax
from jax.experimental import pallas as pl
from jax.experimental.pallas import tpu as pltpu

N_LAYERS = 3
MASKS = (1, 3, 4)
CH = 4
N_EX = N_LAYERS * len(MASKS) * CH

_sem_signal = getattr(pltpu, "semaphore_signal", None) or getattr(pl, "semaphore_signal")
_sem_wait = getattr(pltpu, "semaphore_wait", None) or getattr(pl, "semaphore_wait")
_DeviceIdType = getattr(pl, "DeviceIdType", None) or getattr(pltpu, "DeviceIdType")
_CompilerParams = getattr(pltpu, "CompilerParams", None) or getattr(
    pltpu, "TPUCompilerParams"
)


def kernel(x, Win0, Wout0, Win1, Wout1, Win2, Wout2):
    b, d = x.shape
    cw = d // CH

    def body(
        x_ref,
        win0_ref,
        wout0_ref,
        win1_ref,
        wout1_ref,
        win2_ref,
        wout2_ref,
        out_ref,
        sb,
        rv,
        send_sems,
        recv_sems,
    ):
        my = lax.axis_index("i")

        barrier = pltpu.get_barrier_semaphore()
        for mask in MASKS:
            _sem_signal(
                barrier,
                inc=1,
                device_id=(my ^ mask,),
                device_id_type=_DeviceIdType.MESH,
            )
        _sem_wait(barrier, len(MASKS))

        wins = (win0_ref, win1_ref, win2_ref)
        wouts = (wout0_ref, wout1_ref, wout2_ref)
        bf16 = jnp.bfloat16

        def make_rdma(l, s, c):
            idx = (l * len(MASKS) + s) * CH + c
            return idx, pltpu.make_async_remote_copy(
                src_ref=sb.at[idx],
                dst_ref=rv.at[idx],
                send_sem=send_sems.at[idx],
                recv_sem=recv_sems.at[idx],
                device_id=(my ^ MASKS[s],),
                device_id_type=_DeviceIdType.MESH,
            )

        rdmas = {}

        h = jnp.maximum(
            jnp.dot(
                x_ref[:, :].astype(bf16),
                win0_ref[:, :].astype(bf16),
                preferred_element_type=jnp.float32,
            ),
            0.0,
        )
        for l in range(N_LAYERS):
            hb = h.astype(bf16)
            woutb = wouts[l][:, :].astype(bf16)
            ps = []
            for c in range(CH):
                p = jnp.dot(
                    hb,
                    woutb[:, c * cw : (c + 1) * cw],
                    preferred_element_type=jnp.float32,
                )
                ps.append(p)
                idx, rdma = make_rdma(l, 0, c)
                sb[idx, :, :] = p.astype(bf16)
                rdmas[idx] = rdma
                rdma.start()

            winb_next = (
                wins[l + 1][:, :].astype(bf16) if l + 1 < N_LAYERS else None
            )

            hn = None
            for s in range(len(MASKS)):
                for c in range(CH):
                    idx = (l * len(MASKS) + s) * CH + c
                    rdmas[idx].wait_recv()
                    ps[c] = ps[c] + rv[idx, :, :].astype(jnp.float32)
                    if s + 1 < len(MASKS):
                        idx2, rdma2 = make_rdma(l, s + 1, c)
                        sb[idx2, :, :] = ps[c].astype(bf16)
                        rdmas[idx2] = rdma2
                        rdma2.start()
                    elif l + 1 < N_LAYERS:
                        contrib = jnp.dot(
                            ps[c].astype(bf16),
                            winb_next[c * cw : (c + 1) * cw, :],
                            preferred_element_type=jnp.float32,
                        )
                        hn = contrib if hn is None else hn + contrib
                    else:
                        out_ref[:, c * cw : (c + 1) * cw] = ps[c]
            if l + 1 < N_LAYERS:
                h = jnp.maximum(hn, 0.0)

        for idx, rdma in rdmas.items():
            rdma.wait_send()

    return pl.pallas_call(
        body,
        out_shape=jax.ShapeDtypeStruct((b, d), jnp.float32),
        in_specs=[pl.BlockSpec(memory_space=pltpu.VMEM)] * 7,
        out_specs=pl.BlockSpec(memory_space=pltpu.VMEM),
        scratch_shapes=[
            pltpu.VMEM((N_EX, b, cw), jnp.bfloat16),
            pltpu.VMEM((N_EX, b, cw), jnp.bfloat16),
            pltpu.SemaphoreType.DMA((N_EX,)),
            pltpu.SemaphoreType.DMA((N_EX,)),
        ],
        compiler_params=_CompilerParams(
            collective_id=0, vmem_limit_bytes=100 * 1024 * 1024
        ),
    )(x, Win0, Wout0, Win1, Wout1, Win2, Wout2)
